# baseline (speedup 1.0000x reference)
import functools

import jax
import jax.numpy as jnp
from jax import lax
from jax.experimental import pallas as pl
from jax.experimental.pallas import tpu as pltpu
from jax.experimental.pallas import tpu_sc as plsc

_BM = 128
_RB = 256
_NW = 32
_NC = 2


def _router_body(x_ref, wg_ref, val_ref, idx_ref):
    l = jnp.dot(x_ref[...], wg_ref[...], preferred_element_type=jnp.float32)
    m = jnp.max(l, axis=-1, keepdims=True)
    el = jnp.exp(l - m)
    probs = el / jnp.sum(el, axis=-1, keepdims=True)
    ncols = probs.shape[-1]
    iota = lax.broadcasted_iota(jnp.int32, probs.shape, 1)
    v1 = jnp.max(probs, axis=-1, keepdims=True)
    i1 = jnp.min(jnp.where(probs == v1, iota, ncols), axis=-1, keepdims=True)
    p2 = jnp.where(iota == i1, -1.0, probs)
    v2 = jnp.max(p2, axis=-1, keepdims=True)
    i2 = jnp.min(jnp.where(p2 == v2, iota, ncols), axis=-1, keepdims=True)
    val_ref[...] = jnp.concatenate([v1, v2], axis=-1)
    idx_ref[...] = jnp.concatenate([i1, i2], axis=-1)


def _router(xf, Wg):
    T, H = xf.shape
    E = Wg.shape[1]
    grid = (T // _RB,)
    return pl.pallas_call(
        _router_body,
        grid=grid,
        in_specs=[
            pl.BlockSpec((_RB, H), lambda t: (t, 0)),
            pl.BlockSpec((H, E), lambda t: (0, 0)),
        ],
        out_specs=[
            pl.BlockSpec((_RB, 2), lambda t: (t, 0)),
            pl.BlockSpec((_RB, 2), lambda t: (t, 0)),
        ],
        out_shape=[
            jax.ShapeDtypeStruct((T, 2), jnp.float32),
            jax.ShapeDtypeStruct((T, 2), jnp.int32),
        ],
    )(xf, Wg)


_AB = 512


def _rank_body(a_ref, rank_ref, counts_ref, carry_ref, tri_ref):
    g = pl.program_id(0)

    @pl.when(g == 0)
    def _init():
        carry_ref[...] = jnp.zeros_like(carry_ref)
        r = lax.broadcasted_iota(jnp.int32, (_AB, _AB), 0)
        c = lax.broadcasted_iota(jnp.int32, (_AB, _AB), 1)
        tri_ref[...] = (c < r).astype(jnp.float32)

    a = a_ref[...]
    ncols = carry_ref.shape[-1]
    cols = lax.broadcasted_iota(jnp.int32, (_AB, ncols), 1)
    onehot = (a == cols).astype(jnp.float32)
    prior = jnp.dot(tri_ref[...], onehot, preferred_element_type=jnp.float32)
    carry = carry_ref[0:1, :].astype(jnp.float32)
    rank = jnp.sum(onehot * (prior + carry), axis=1, keepdims=True)
    rank_ref[...] = rank.astype(jnp.int32)
    new_counts = carry + jnp.sum(onehot, axis=0, keepdims=True)
    carry_ref[0:1, :] = new_counts.astype(jnp.int32)
    counts_ref[...] = new_counts.astype(jnp.int32)


def _rank_kernel(a, E):
    A = a.shape[0]
    grid = (A // _AB,)
    rank, counts = pl.pallas_call(
        _rank_body,
        grid=grid,
        in_specs=[pl.BlockSpec((_AB, 1), lambda g: (g, 0))],
        out_specs=[
            pl.BlockSpec((_AB, 1), lambda g: (g, 0)),
            pl.BlockSpec((1, E), lambda g: (0, 0)),
        ],
        out_shape=[
            jax.ShapeDtypeStruct((A, 1), jnp.int32),
            jax.ShapeDtypeStruct((1, E), jnp.int32),
        ],
        scratch_shapes=[
            pltpu.VMEM((8, E), jnp.int32),
            pltpu.VMEM((_AB, _AB), jnp.float32),
        ],
        compiler_params=pltpu.CompilerParams(
            dimension_semantics=("arbitrary",)),
    )(a.reshape(A, 1))
    return rank[:, 0], counts[0]


def _dispatch(idx, val, E, nblk):
    T, K = idx.shape
    a = idx.reshape(-1)
    rank, counts = _rank_kernel(a, E)
    blocks_pe = (counts + _BM - 1) // _BM
    bends = jnp.cumsum(blocks_pe)
    bstart = bends - blocks_pe
    ppos = bstart[a] * _BM + rank
    pos = ppos.reshape(T, K)
    tpw = T // _NW
    ppos3 = pos.reshape(_NW, tpw, K).transpose(0, 2, 1)
    val3 = val.reshape(_NW, tpw, K).transpose(0, 2, 1)
    g_ids = jnp.arange(nblk, dtype=jnp.int32)
    block_expert = jnp.minimum(
        jnp.sum((bends[None, :] <= g_ids[:, None]).astype(jnp.int32), axis=1),
        E - 1).astype(jnp.int32)
    return ppos3, val3, pos[:, 0], pos[:, 1], block_expert


def _sc_dispatch(xf, ppos3, val3, npad):
    T, H = xf.shape
    NW, K, tpw = ppos3.shape
    mesh = plsc.VectorSubcoreMesh(core_axis_name="c", subcore_axis_name="s")

    @functools.partial(
        pl.kernel, mesh=mesh,
        out_type=[
            jax.ShapeDtypeStruct((npad, H), jnp.float32),
            jax.ShapeDtypeStruct((npad,), jnp.float32),
        ],
        scratch_types=[
            pltpu.VMEM((tpw, H), jnp.float32),
            pltpu.VMEM((K, tpw), jnp.int32),
            pltpu.VMEM((K, tpw), jnp.float32),
            [pltpu.SemaphoreType.DMA] * 4,
        ],
    )
    def k(xf_hbm, pp_hbm, v_hbm, xs_hbm, rw_hbm, xbuf, ibuf, vbuf, sems):
        wid = lax.axis_index("s") * _NC + lax.axis_index("c")
        pltpu.sync_copy(pp_hbm.at[wid], ibuf)
        pltpu.sync_copy(v_hbm.at[wid], vbuf)
        pltpu.sync_copy(xf_hbm.at[pl.ds(wid * tpw, tpw)], xbuf)
        hs = [
            pltpu.async_copy(xbuf, xs_hbm.at[ibuf.at[0]], sems[0]),
            pltpu.async_copy(xbuf, xs_hbm.at[ibuf.at[1]], sems[1]),
            pltpu.async_copy(vbuf.at[0], rw_hbm.at[ibuf.at[0]], sems[2]),
            pltpu.async_copy(vbuf.at[1], rw_hbm.at[ibuf.at[1]], sems[3]),
        ]
        for h in hs:
            h.wait()

    return k(xf, ppos3, val3)


def _ffn_body(be_ref, xs_ref, w1_ref, b1_ref, w2_ref, b2_ref, rw_ref, out_ref):
    x = xs_ref[...]
    h = jnp.dot(x, w1_ref[0], preferred_element_type=jnp.float32) + b1_ref[0, 0]
    h = h * jax.nn.sigmoid(h)
    y = jnp.dot(h, w2_ref[0], preferred_element_type=jnp.float32) + b2_ref[0, 0]
    out_ref[...] = y * rw_ref[...]


def _ffn(block_expert, xs, W1, b1, W2, b2, rw):
    E, H, I2 = W1.shape
    npad = xs.shape[0]
    nblk = npad // _BM
    grid_spec = pltpu.PrefetchScalarGridSpec(
        num_scalar_prefetch=1,
        grid=(nblk,),
        in_specs=[
            pl.BlockSpec((_BM, H), lambda g, be: (g, 0)),
            pl.BlockSpec((1, H, I2), lambda g, be: (be[g], 0, 0)),
            pl.BlockSpec((1, 1, I2), lambda g, be: (be[g], 0, 0)),
            pl.BlockSpec((1, I2, H), lambda g, be: (be[g], 0, 0)),
            pl.BlockSpec((1, 1, H), lambda g, be: (be[g], 0, 0)),
            pl.BlockSpec((_BM, 1), lambda g, be: (g, 0)),
        ],
        out_specs=pl.BlockSpec((_BM, H), lambda g, be: (g, 0)),
    )
    return pl.pallas_call(
        _ffn_body,
        grid_spec=grid_spec,
        out_shape=jax.ShapeDtypeStruct((npad, H), jnp.float32),
        compiler_params=pltpu.CompilerParams(
            dimension_semantics=("arbitrary",)),
    )(block_expert, xs, W1, b1.reshape(E, 1, I2), W2, b2.reshape(E, 1, H),
      rw.reshape(npad, 1))


def _sc_combine(ys, pos0, pos1):
    npad, H = ys.shape
    T = pos0.shape[0]
    per_w = T // _NW
    ch = 64
    nch = per_w // ch
    nvec = ch * H // 16
    hv = H // 16
    mesh = plsc.VectorSubcoreMesh(core_axis_name="c", subcore_axis_name="s")

    @functools.partial(
        pl.kernel, mesh=mesh,
        out_type=jax.ShapeDtypeStruct((T, H), jnp.float32),
        scratch_types=[
            pltpu.VMEM((ch,), jnp.int32),
            pltpu.VMEM((ch,), jnp.int32),
            pltpu.VMEM((ch, H), jnp.float32),
            pltpu.VMEM((ch, H), jnp.float32),
            pltpu.SemaphoreType.DMA,
            pltpu.SemaphoreType.DMA,
        ],
    )
    def k(ys_hbm, p0_hbm, p1_hbm, out_hbm, i0_v, i1_v, buf_a, buf_b, sem_a, sem_b):
        wid = lax.axis_index("s") * _NC + lax.axis_index("c")
        base = wid * per_w

        def chunk(c, carry):
            off = base + c * ch
            pltpu.sync_copy(p0_hbm.at[pl.ds(off, ch)], i0_v)
            pltpu.sync_copy(p1_hbm.at[pl.ds(off, ch)], i1_v)
            cp_a = pltpu.async_copy(ys_hbm.at[i0_v], buf_a, sem_a)
            cp_b = pltpu.async_copy(ys_hbm.at[i1_v], buf_b, sem_b)
            cp_a.wait()
            cp_b.wait()

            def add16(j, cc):
                r = j // hv
                col = (j % hv) * 16
                buf_a[r, pl.ds(col, 16)] = (
                    buf_a[r, pl.ds(col, 16)] + buf_b[r, pl.ds(col, 16)])
                return cc

            lax.fori_loop(0, nvec, add16, 0)
            pltpu.sync_copy(buf_a, out_hbm.at[pl.ds(off, ch)])
            return carry

        lax.fori_loop(0, nch, chunk, 0)

    return k(ys, pos0, pos1)


def kernel(x, Wg, W1, b1, W2, b2):
    b, s, h = x.shape
    E = Wg.shape[1]
    K = 2
    xf = x.reshape(-1, h)
    T = xf.shape[0]
    nblk = (T * K) // _BM + E
    val, idx = _router(xf, Wg)
    ppos3, val3, pos0, pos1, block_expert = _dispatch(idx, val, E, nblk)
    xs, rw = _sc_dispatch(xf, ppos3, val3, nblk * _BM)
    ys = _ffn(block_expert, xs, W1, b1, W2, b2, rw)
    out = _sc_combine(ys, pos0, pos1)
    return out.reshape(b, s, h)


# speedup vs baseline: 1.0309x; 1.0309x over previous
import functools

import jax
import jax.numpy as jnp
from jax import lax
from jax.experimental import pallas as pl
from jax.experimental.pallas import tpu as pltpu
from jax.experimental.pallas import tpu_sc as plsc

_BM = 128
_RB = 256
_NW = 32
_NC = 2


def _router_body(x_ref, wg_ref, val_ref, idx_ref, rank_ref, counts_ref,
                 carry_ref, tri_ref):
    g = pl.program_id(0)
    nab = 2 * _RB

    @pl.when(g == 0)
    def _init():
        carry_ref[...] = jnp.zeros_like(carry_ref)
        r = lax.broadcasted_iota(jnp.int32, (nab, nab), 0)
        c = lax.broadcasted_iota(jnp.int32, (nab, nab), 1)
        tri_ref[...] = (c < r).astype(jnp.float32)

    l = jnp.dot(x_ref[...], wg_ref[...], preferred_element_type=jnp.float32)
    m = jnp.max(l, axis=-1, keepdims=True)
    el = jnp.exp(l - m)
    probs = el / jnp.sum(el, axis=-1, keepdims=True)
    ncols = probs.shape[-1]
    iota = lax.broadcasted_iota(jnp.int32, probs.shape, 1)
    v1 = jnp.max(probs, axis=-1, keepdims=True)
    i1 = jnp.min(jnp.where(probs == v1, iota, ncols), axis=-1, keepdims=True)
    p2 = jnp.where(iota == i1, -1.0, probs)
    v2 = jnp.max(p2, axis=-1, keepdims=True)
    i2 = jnp.min(jnp.where(p2 == v2, iota, ncols), axis=-1, keepdims=True)
    val_ref[...] = jnp.concatenate([v1, v2], axis=-1)
    idx_ref[...] = jnp.concatenate([i1, i2], axis=-1)
    ncols_e = counts_ref.shape[-1]
    cols1 = lax.broadcasted_iota(jnp.int32, (_RB, ncols_e), 1)
    onehot = jnp.concatenate(
        [(i1 == cols1).astype(jnp.float32), (i2 == cols1).astype(jnp.float32)],
        axis=0)
    prior = jnp.dot(tri_ref[...], onehot, preferred_element_type=jnp.float32)
    carry = carry_ref[0:1, :].astype(jnp.float32)
    rank = jnp.sum(onehot * (prior + carry), axis=1, keepdims=True)
    rank_ref[...] = rank.astype(jnp.int32)
    new_counts = carry + jnp.sum(onehot, axis=0, keepdims=True)
    carry_ref[0:1, :] = new_counts.astype(jnp.int32)
    counts_ref[...] = new_counts.astype(jnp.int32)


def _router(xf, Wg):
    T, H = xf.shape
    E = Wg.shape[1]
    grid = (T // _RB,)
    nab = 2 * _RB
    val, idx, rank, counts = pl.pallas_call(
        _router_body,
        grid=grid,
        in_specs=[
            pl.BlockSpec((_RB, H), lambda t: (t, 0)),
            pl.BlockSpec((H, E), lambda t: (0, 0)),
        ],
        out_specs=[
            pl.BlockSpec((_RB, 2), lambda t: (t, 0)),
            pl.BlockSpec((_RB, 2), lambda t: (t, 0)),
            pl.BlockSpec((nab, 1), lambda t: (t, 0)),
            pl.BlockSpec((1, E), lambda t: (0, 0)),
        ],
        out_shape=[
            jax.ShapeDtypeStruct((T, 2), jnp.float32),
            jax.ShapeDtypeStruct((T, 2), jnp.int32),
            jax.ShapeDtypeStruct((2 * T, 1), jnp.int32),
            jax.ShapeDtypeStruct((1, E), jnp.int32),
        ],
        scratch_shapes=[
            pltpu.VMEM((8, E), jnp.int32),
            pltpu.VMEM((nab, nab), jnp.float32),
        ],
        compiler_params=pltpu.CompilerParams(
            dimension_semantics=("arbitrary",)),
    )(xf, Wg)
    return val, idx, rank[:, 0], counts[0]


def _dispatch(idx, val, rank, counts, nblk):
    T, K = idx.shape
    E = counts.shape[0]
    blocks_pe = (counts + _BM - 1) // _BM
    bends = jnp.cumsum(blocks_pe)
    bstart = bends - blocks_pe
    rank_tk = rank.reshape(T // _RB, K, _RB).transpose(0, 2, 1).reshape(T, K)
    pos = bstart[idx] * _BM + rank_tk
    tpw = T // _NW
    ppos3 = pos.reshape(_NW, tpw, K).transpose(0, 2, 1)
    val3 = val.reshape(_NW, tpw, K).transpose(0, 2, 1)
    g_ids = jnp.arange(nblk, dtype=jnp.int32)
    block_expert = jnp.minimum(
        jnp.sum((bends[None, :] <= g_ids[:, None]).astype(jnp.int32), axis=1),
        E - 1).astype(jnp.int32)
    return ppos3, val3, pos[:, 0], pos[:, 1], block_expert


def _sc_dispatch(xf, ppos3, val3, npad):
    T, H = xf.shape
    NW, K, tpw = ppos3.shape
    mesh = plsc.VectorSubcoreMesh(core_axis_name="c", subcore_axis_name="s")

    @functools.partial(
        pl.kernel, mesh=mesh,
        out_type=[
            jax.ShapeDtypeStruct((npad, H), jnp.float32),
            jax.ShapeDtypeStruct((npad,), jnp.float32),
        ],
        scratch_types=[
            pltpu.VMEM((tpw, H), jnp.float32),
            pltpu.VMEM((K, tpw), jnp.int32),
            pltpu.VMEM((K, tpw), jnp.float32),
            [pltpu.SemaphoreType.DMA] * 4,
        ],
    )
    def k(xf_hbm, pp_hbm, v_hbm, xs_hbm, rw_hbm, xbuf, ibuf, vbuf, sems):
        wid = lax.axis_index("s") * _NC + lax.axis_index("c")
        pltpu.sync_copy(pp_hbm.at[wid], ibuf)
        pltpu.sync_copy(v_hbm.at[wid], vbuf)
        pltpu.sync_copy(xf_hbm.at[pl.ds(wid * tpw, tpw)], xbuf)
        hs = [
            pltpu.async_copy(xbuf, xs_hbm.at[ibuf.at[0]], sems[0]),
            pltpu.async_copy(xbuf, xs_hbm.at[ibuf.at[1]], sems[1]),
            pltpu.async_copy(vbuf.at[0], rw_hbm.at[ibuf.at[0]], sems[2]),
            pltpu.async_copy(vbuf.at[1], rw_hbm.at[ibuf.at[1]], sems[3]),
        ]
        for h in hs:
            h.wait()

    return k(xf, ppos3, val3)


def _ffn_body(be_ref, xs_ref, w1_ref, b1_ref, w2_ref, b2_ref, rw_ref, out_ref):
    x = xs_ref[...]
    h = jnp.dot(x, w1_ref[0], preferred_element_type=jnp.float32) + b1_ref[0, 0]
    h = h * jax.nn.sigmoid(h)
    y = jnp.dot(h, w2_ref[0], preferred_element_type=jnp.float32) + b2_ref[0, 0]
    out_ref[...] = y * rw_ref[...]


def _ffn(block_expert, xs, W1, b1, W2, b2, rw):
    E, H, I2 = W1.shape
    npad = xs.shape[0]
    nblk = npad // _BM
    grid_spec = pltpu.PrefetchScalarGridSpec(
        num_scalar_prefetch=1,
        grid=(nblk,),
        in_specs=[
            pl.BlockSpec((_BM, H), lambda g, be: (g, 0)),
            pl.BlockSpec((1, H, I2), lambda g, be: (be[g], 0, 0)),
            pl.BlockSpec((1, 1, I2), lambda g, be: (be[g], 0, 0)),
            pl.BlockSpec((1, I2, H), lambda g, be: (be[g], 0, 0)),
            pl.BlockSpec((1, 1, H), lambda g, be: (be[g], 0, 0)),
            pl.BlockSpec((_BM, 1), lambda g, be: (g, 0)),
        ],
        out_specs=pl.BlockSpec((_BM, H), lambda g, be: (g, 0)),
    )
    return pl.pallas_call(
        _ffn_body,
        grid_spec=grid_spec,
        out_shape=jax.ShapeDtypeStruct((npad, H), jnp.float32),
        compiler_params=pltpu.CompilerParams(
            dimension_semantics=("arbitrary",)),
    )(block_expert, xs, W1, b1.reshape(E, 1, I2), W2, b2.reshape(E, 1, H),
      rw.reshape(npad, 1))


def _sc_combine(ys, pos0, pos1):
    npad, H = ys.shape
    T = pos0.shape[0]
    per_w = T // _NW
    ch = 64
    nch = per_w // ch
    nvec = ch * H // 16
    hv = H // 16
    mesh = plsc.VectorSubcoreMesh(core_axis_name="c", subcore_axis_name="s")

    @functools.partial(
        pl.kernel, mesh=mesh,
        out_type=jax.ShapeDtypeStruct((T, H), jnp.float32),
        scratch_types=[
            pltpu.VMEM((ch,), jnp.int32),
            pltpu.VMEM((ch,), jnp.int32),
            pltpu.VMEM((ch, H), jnp.float32),
            pltpu.VMEM((ch, H), jnp.float32),
            pltpu.SemaphoreType.DMA,
            pltpu.SemaphoreType.DMA,
        ],
    )
    def k(ys_hbm, p0_hbm, p1_hbm, out_hbm, i0_v, i1_v, buf_a, buf_b, sem_a, sem_b):
        wid = lax.axis_index("s") * _NC + lax.axis_index("c")
        base = wid * per_w

        def chunk(c, carry):
            off = base + c * ch
            pltpu.sync_copy(p0_hbm.at[pl.ds(off, ch)], i0_v)
            pltpu.sync_copy(p1_hbm.at[pl.ds(off, ch)], i1_v)
            cp_a = pltpu.async_copy(ys_hbm.at[i0_v], buf_a, sem_a)
            cp_b = pltpu.async_copy(ys_hbm.at[i1_v], buf_b, sem_b)
            cp_a.wait()
            cp_b.wait()

            def add16(j, cc):
                r = j // hv
                col = (j % hv) * 16
                buf_a[r, pl.ds(col, 16)] = (
                    buf_a[r, pl.ds(col, 16)] + buf_b[r, pl.ds(col, 16)])
                return cc

            lax.fori_loop(0, nvec, add16, 0)
            pltpu.sync_copy(buf_a, out_hbm.at[pl.ds(off, ch)])
            return carry

        lax.fori_loop(0, nch, chunk, 0)

    return k(ys, pos0, pos1)


def kernel(x, Wg, W1, b1, W2, b2):
    b, s, h = x.shape
    E = Wg.shape[1]
    K = 2
    xf = x.reshape(-1, h)
    T = xf.shape[0]
    nblk = (T * K) // _BM + E
    val, idx, rank, counts = _router(xf, Wg)
    ppos3, val3, pos0, pos1, block_expert = _dispatch(idx, val, rank, counts, nblk)
    xs, rw = _sc_dispatch(xf, ppos3, val3, nblk * _BM)
    ys = _ffn(block_expert, xs, W1, b1, W2, b2, rw)
    out = _sc_combine(ys, pos0, pos1)
    return out.reshape(b, s, h)


# speedup vs baseline: 1.0559x; 1.0243x over previous
import functools

import jax
import jax.numpy as jnp
from jax import lax
from jax.experimental import pallas as pl
from jax.experimental.pallas import tpu as pltpu
from jax.experimental.pallas import tpu_sc as plsc

_BM = 128
_RB = 256
_NW = 32
_NC = 2


def _router_body(x_ref, wg_ref, val_ref, idx_ref, rank_ref, counts_ref,
                 carry_ref, tri_ref):
    g = pl.program_id(0)
    nab = 2 * _RB

    @pl.when(g == 0)
    def _init():
        carry_ref[...] = jnp.zeros_like(carry_ref)
        r = lax.broadcasted_iota(jnp.int32, (nab, nab), 0)
        c = lax.broadcasted_iota(jnp.int32, (nab, nab), 1)
        tri_ref[...] = (c < r).astype(jnp.float32)

    l = jnp.dot(x_ref[...], wg_ref[...], preferred_element_type=jnp.float32)
    m = jnp.max(l, axis=-1, keepdims=True)
    el = jnp.exp(l - m)
    probs = el / jnp.sum(el, axis=-1, keepdims=True)
    ncols = probs.shape[-1]
    iota = lax.broadcasted_iota(jnp.int32, probs.shape, 1)
    v1 = jnp.max(probs, axis=-1, keepdims=True)
    i1 = jnp.min(jnp.where(probs == v1, iota, ncols), axis=-1, keepdims=True)
    p2 = jnp.where(iota == i1, -1.0, probs)
    v2 = jnp.max(p2, axis=-1, keepdims=True)
    i2 = jnp.min(jnp.where(p2 == v2, iota, ncols), axis=-1, keepdims=True)
    val_ref[...] = jnp.concatenate([v1, v2], axis=-1)
    idx_ref[...] = jnp.concatenate([i1, i2], axis=-1)
    ncols_e = counts_ref.shape[-1]
    cols1 = lax.broadcasted_iota(jnp.int32, (_RB, ncols_e), 1)
    onehot = jnp.concatenate(
        [(i1 == cols1).astype(jnp.float32), (i2 == cols1).astype(jnp.float32)],
        axis=0)
    prior = jnp.dot(tri_ref[...], onehot, preferred_element_type=jnp.float32)
    carry = carry_ref[0:1, :].astype(jnp.float32)
    rank = jnp.sum(onehot * (prior + carry), axis=1, keepdims=True)
    rank_ref[...] = rank.astype(jnp.int32)
    new_counts = carry + jnp.sum(onehot, axis=0, keepdims=True)
    carry_ref[0:1, :] = new_counts.astype(jnp.int32)
    counts_ref[...] = new_counts.astype(jnp.int32)


def _router(xf, Wg):
    T, H = xf.shape
    E = Wg.shape[1]
    grid = (T // _RB,)
    nab = 2 * _RB
    val, idx, rank, counts = pl.pallas_call(
        _router_body,
        grid=grid,
        in_specs=[
            pl.BlockSpec((_RB, H), lambda t: (t, 0)),
            pl.BlockSpec((H, E), lambda t: (0, 0)),
        ],
        out_specs=[
            pl.BlockSpec((_RB, 2), lambda t: (t, 0)),
            pl.BlockSpec((_RB, 2), lambda t: (t, 0)),
            pl.BlockSpec((nab, 1), lambda t: (t, 0)),
            pl.BlockSpec((1, E), lambda t: (0, 0)),
        ],
        out_shape=[
            jax.ShapeDtypeStruct((T, 2), jnp.float32),
            jax.ShapeDtypeStruct((T, 2), jnp.int32),
            jax.ShapeDtypeStruct((2 * T, 1), jnp.int32),
            jax.ShapeDtypeStruct((1, E), jnp.int32),
        ],
        scratch_shapes=[
            pltpu.VMEM((8, E), jnp.int32),
            pltpu.VMEM((nab, nab), jnp.float32),
        ],
        compiler_params=pltpu.CompilerParams(
            dimension_semantics=("arbitrary",)),
    )(xf, Wg)
    return val, idx, rank[:, 0], counts[0]


def _dispatch(idx, val, rank, counts, nblk):
    T, K = idx.shape
    E = counts.shape[0]
    blocks_pe = (counts + _BM - 1) // _BM
    bends = jnp.cumsum(blocks_pe)
    bstart = bends - blocks_pe
    rank_tk = rank.reshape(T // _RB, K, _RB).transpose(0, 2, 1).reshape(T, K)
    pos = bstart[idx] * _BM + rank_tk
    tpw = T // _NW
    ppos3 = pos.reshape(_NW, tpw, K).transpose(0, 2, 1)
    val3 = val.reshape(_NW, tpw, K).transpose(0, 2, 1)
    g_ids = jnp.arange(nblk, dtype=jnp.int32)
    block_expert = jnp.minimum(
        jnp.sum((bends[None, :] <= g_ids[:, None]).astype(jnp.int32), axis=1),
        E - 1).astype(jnp.int32)
    block_expert = jnp.concatenate([block_expert, bends[-1:]])
    return ppos3, val3, pos[:, 0], pos[:, 1], block_expert


def _sc_dispatch(xf, ppos3, val3, npad):
    T, H = xf.shape
    NW, K, tpw = ppos3.shape
    mesh = plsc.VectorSubcoreMesh(core_axis_name="c", subcore_axis_name="s")

    @functools.partial(
        pl.kernel, mesh=mesh,
        out_type=[
            jax.ShapeDtypeStruct((npad, H), jnp.float32),
            jax.ShapeDtypeStruct((npad,), jnp.float32),
        ],
        scratch_types=[
            pltpu.VMEM((tpw, H), jnp.float32),
            pltpu.VMEM((K, tpw), jnp.int32),
            pltpu.VMEM((K, tpw), jnp.float32),
            [pltpu.SemaphoreType.DMA] * 4,
        ],
    )
    def k(xf_hbm, pp_hbm, v_hbm, xs_hbm, rw_hbm, xbuf, ibuf, vbuf, sems):
        wid = lax.axis_index("s") * _NC + lax.axis_index("c")
        pltpu.sync_copy(pp_hbm.at[wid], ibuf)
        pltpu.sync_copy(v_hbm.at[wid], vbuf)
        pltpu.sync_copy(xf_hbm.at[pl.ds(wid * tpw, tpw)], xbuf)
        hs = [
            pltpu.async_copy(xbuf, xs_hbm.at[ibuf.at[0]], sems[0]),
            pltpu.async_copy(xbuf, xs_hbm.at[ibuf.at[1]], sems[1]),
            pltpu.async_copy(vbuf.at[0], rw_hbm.at[ibuf.at[0]], sems[2]),
            pltpu.async_copy(vbuf.at[1], rw_hbm.at[ibuf.at[1]], sems[3]),
        ]
        for h in hs:
            h.wait()

    return k(xf, ppos3, val3)


def _ffn_body(be_ref, xs_ref, w1_ref, b1_ref, w2_ref, b2_ref, rw_ref, out_ref):
    nblk = be_ref.shape[0] - 1

    @pl.when(pl.program_id(0) < be_ref[nblk])
    def _compute():
        x = xs_ref[...]
        h = (jnp.dot(x, w1_ref[0], preferred_element_type=jnp.float32)
             + b1_ref[0, 0])
        h = h * jax.nn.sigmoid(h)
        y = (jnp.dot(h, w2_ref[0], preferred_element_type=jnp.float32)
             + b2_ref[0, 0])
        out_ref[...] = y * rw_ref[...]


def _ffn(block_expert, xs, W1, b1, W2, b2, rw):
    E, H, I2 = W1.shape
    npad = xs.shape[0]
    nblk = npad // _BM
    grid_spec = pltpu.PrefetchScalarGridSpec(
        num_scalar_prefetch=1,
        grid=(nblk,),
        in_specs=[
            pl.BlockSpec((_BM, H), lambda g, be: (g, 0)),
            pl.BlockSpec((1, H, I2), lambda g, be: (be[g], 0, 0)),
            pl.BlockSpec((1, 1, I2), lambda g, be: (be[g], 0, 0)),
            pl.BlockSpec((1, I2, H), lambda g, be: (be[g], 0, 0)),
            pl.BlockSpec((1, 1, H), lambda g, be: (be[g], 0, 0)),
            pl.BlockSpec((_BM, 1), lambda g, be: (g, 0)),
        ],
        out_specs=pl.BlockSpec((_BM, H), lambda g, be: (g, 0)),
    )
    return pl.pallas_call(
        _ffn_body,
        grid_spec=grid_spec,
        out_shape=jax.ShapeDtypeStruct((npad, H), jnp.float32),
        compiler_params=pltpu.CompilerParams(
            dimension_semantics=("arbitrary",)),
    )(block_expert, xs, W1, b1.reshape(E, 1, I2), W2, b2.reshape(E, 1, H),
      rw.reshape(npad, 1))


def _sc_combine(ys, pos0, pos1):
    npad, H = ys.shape
    T = pos0.shape[0]
    per_w = T // _NW
    ch = 64
    nch = per_w // ch
    nvec = ch * H // 16
    hv = H // 16
    mesh = plsc.VectorSubcoreMesh(core_axis_name="c", subcore_axis_name="s")

    @functools.partial(
        pl.kernel, mesh=mesh,
        out_type=jax.ShapeDtypeStruct((T, H), jnp.float32),
        scratch_types=[
            pltpu.VMEM((ch,), jnp.int32),
            pltpu.VMEM((ch,), jnp.int32),
            pltpu.VMEM((ch, H), jnp.float32),
            pltpu.VMEM((ch, H), jnp.float32),
            pltpu.SemaphoreType.DMA,
            pltpu.SemaphoreType.DMA,
        ],
    )
    def k(ys_hbm, p0_hbm, p1_hbm, out_hbm, i0_v, i1_v, buf_a, buf_b, sem_a, sem_b):
        wid = lax.axis_index("s") * _NC + lax.axis_index("c")
        base = wid * per_w

        def chunk(c, carry):
            off = base + c * ch
            pltpu.sync_copy(p0_hbm.at[pl.ds(off, ch)], i0_v)
            pltpu.sync_copy(p1_hbm.at[pl.ds(off, ch)], i1_v)
            cp_a = pltpu.async_copy(ys_hbm.at[i0_v], buf_a, sem_a)
            cp_b = pltpu.async_copy(ys_hbm.at[i1_v], buf_b, sem_b)
            cp_a.wait()
            cp_b.wait()

            def add16(j, cc):
                r = j // hv
                col = (j % hv) * 16
                buf_a[r, pl.ds(col, 16)] = (
                    buf_a[r, pl.ds(col, 16)] + buf_b[r, pl.ds(col, 16)])
                return cc

            lax.fori_loop(0, nvec, add16, 0)
            pltpu.sync_copy(buf_a, out_hbm.at[pl.ds(off, ch)])
            return carry

        lax.fori_loop(0, nch, chunk, 0)

    return k(ys, pos0, pos1)


def kernel(x, Wg, W1, b1, W2, b2):
    b, s, h = x.shape
    E = Wg.shape[1]
    K = 2
    xf = x.reshape(-1, h)
    T = xf.shape[0]
    nblk = (T * K) // _BM + E
    val, idx, rank, counts = _router(xf, Wg)
    ppos3, val3, pos0, pos1, block_expert = _dispatch(idx, val, rank, counts, nblk)
    xs, rw = _sc_dispatch(xf, ppos3, val3, nblk * _BM)
    ys = _ffn(block_expert, xs, W1, b1, W2, b2, rw)
    out = _sc_combine(ys, pos0, pos1)
    return out.reshape(b, s, h)
